# Initial kernel scaffold; baseline (speedup 1.0000x reference)
import jax
import jax.numpy as jnp
from jax.experimental import pallas as pl


def kernel(x0, x1, x2, x3, x4, x5, x6, x7):
    raise NotImplementedError("write your pallas kernel here")


# speedup vs baseline: 156.6890x; 156.6890x over previous
import functools

import numpy as np
import jax
import jax.numpy as jnp
from jax import lax
from jax.experimental import pallas as pl
from jax.experimental.pallas import tpu as pltpu
from jax.experimental.pallas import tpu_sc as plsc

_LENS = (4096, 3584, 3072, 2560, 2048, 1536, 1024, 512)
_D = 256
_TOTAL = sum(_LENS)
_NC = 2
_NS = 16
_NW = _NC * _NS


def _metadata():
    lengths = np.array(_LENS, np.int64)
    max_len = int(lengths.max())
    bs = (lengths[None, :] > np.arange(max_len)[:, None]).sum(axis=1)
    off = np.zeros(max_len, np.int64)
    off[1:] = np.cumsum(bs)[:-1]
    dests = [(off[:L] + j).astype(np.int32) for j, L in enumerate(_LENS)]
    sorted_idx = np.argsort(-lengths, kind="stable")
    unsorted_idx = np.argsort(sorted_idx)
    return bs, sorted_idx, unsorted_idx, dests


_BS, _SORTED, _UNSORTED, _DESTS = _metadata()
_CNTS = tuple(L // _NW for L in _LENS)


def _pack_body(*refs):
    xs = refs[0:8]
    ds = refs[8:16]
    out = refs[16]
    rows_v = refs[17]
    idxs = refs[18:26]
    sem = refs[26]
    wid = lax.axis_index("s") * _NC + lax.axis_index("c")
    for j in range(8):
        cnt = _CNTS[j]
        base = wid * cnt
        pltpu.sync_copy(xs[j].at[pl.ds(base, cnt)], rows_v.at[pl.ds(0, cnt)])
        pltpu.sync_copy(ds[j].at[pl.ds(base, cnt)], idxs[j])
        pltpu.async_copy(rows_v.at[pl.ds(0, cnt)], out.at[idxs[j]], sem).wait()


_pack = functools.partial(
    pl.kernel,
    mesh=plsc.VectorSubcoreMesh(core_axis_name="c", subcore_axis_name="s"),
    out_type=jax.ShapeDtypeStruct((_TOTAL, _D), jnp.float32),
    scratch_types=[pltpu.VMEM((max(_CNTS), _D), jnp.float32)]
    + [pltpu.VMEM((c,), jnp.int32) for c in _CNTS]
    + [pltpu.SemaphoreType.DMA],
)(_pack_body)


def kernel(x0, x1, x2, x3, x4, x5, x6, x7):
    xs = (x0, x1, x2, x3, x4, x5, x6, x7)
    dconsts = tuple(jnp.asarray(d) for d in _DESTS)
    data = _pack(*xs, *dconsts)
    return (
        data,
        jnp.asarray(_BS, dtype=jnp.int64),
        jnp.asarray(_SORTED, dtype=jnp.int64),
        jnp.asarray(_UNSORTED, dtype=jnp.int64),
    )


# speedup vs baseline: 184.5875x; 1.1781x over previous
import functools

import numpy as np
import jax
import jax.numpy as jnp
from jax import lax
from jax.experimental import pallas as pl
from jax.experimental.pallas import tpu as pltpu
from jax.experimental.pallas import tpu_sc as plsc

_LENS = (4096, 3584, 3072, 2560, 2048, 1536, 1024, 512)
_D = 256
_TOTAL = sum(_LENS)
_NC = 2
_NS = 16
_NW = _NC * _NS


def _metadata():
    lengths = np.array(_LENS, np.int64)
    max_len = int(lengths.max())
    bs = (lengths[None, :] > np.arange(max_len)[:, None]).sum(axis=1)
    off = np.zeros(max_len, np.int64)
    off[1:] = np.cumsum(bs)[:-1]
    dests = [(off[:L] + j).astype(np.int32) for j, L in enumerate(_LENS)]
    sorted_idx = np.argsort(-lengths, kind="stable")
    unsorted_idx = np.argsort(sorted_idx)
    return bs, sorted_idx, unsorted_idx, dests


_BS, _SORTED, _UNSORTED, _DESTS = _metadata()
_CNTS = tuple(L // _NW for L in _LENS)


_SLOT_ROWS = tuple(max(_CNTS[b], _CNTS[b + 4]) for b in range(4))


def _pack_body(*refs):
    xs = refs[0:8]
    ds = refs[8:16]
    out = refs[16]
    bufs = refs[17:21]
    idxs = refs[21:29]
    gsem = refs[29:33]
    ssem = refs[33:37]
    isem = refs[37:45]
    wid = lax.axis_index("s") * _NC + lax.axis_index("c")

    def gather(j):
        cnt = _CNTS[j]
        return pltpu.async_copy(
            xs[j].at[pl.ds(wid * cnt, cnt)],
            bufs[j % 4].at[pl.ds(0, cnt)],
            gsem[j % 4],
        )

    ih = [
        pltpu.async_copy(
            ds[j].at[pl.ds(wid * _CNTS[j], _CNTS[j])], idxs[j], isem[j]
        )
        for j in range(8)
    ]
    gh = [gather(0), gather(1), None, None]
    sh = [None] * 4
    for j in range(8):
        b = j % 4
        cnt = _CNTS[j]
        gh[b].wait()
        ih[j].wait()
        sh[b] = pltpu.async_copy(
            bufs[b].at[pl.ds(0, cnt)], out.at[idxs[j]], ssem[b]
        )
        if j + 2 < 8:
            if j >= 2:
                sh[(j - 2) % 4].wait()
            gh[(j + 2) % 4] = gather(j + 2)
    for j in range(4, 8):
        sh[j % 4].wait()


_pack = functools.partial(
    pl.kernel,
    mesh=plsc.VectorSubcoreMesh(core_axis_name="c", subcore_axis_name="s"),
    out_type=jax.ShapeDtypeStruct((_TOTAL, _D), jnp.float32),
    scratch_types=[pltpu.VMEM((r, _D), jnp.float32) for r in _SLOT_ROWS]
    + [pltpu.VMEM((c,), jnp.int32) for c in _CNTS]
    + [pltpu.SemaphoreType.DMA for _ in range(4)]
    + [pltpu.SemaphoreType.DMA for _ in range(4)]
    + [pltpu.SemaphoreType.DMA for _ in range(8)],
)(_pack_body)


def kernel(x0, x1, x2, x3, x4, x5, x6, x7):
    xs = (x0, x1, x2, x3, x4, x5, x6, x7)
    dconsts = tuple(jnp.asarray(d) for d in _DESTS)
    data = _pack(*xs, *dconsts)
    return (
        data,
        jnp.asarray(_BS, dtype=jnp.int64),
        jnp.asarray(_SORTED, dtype=jnp.int64),
        jnp.asarray(_UNSORTED, dtype=jnp.int64),
    )


# speedup vs baseline: 223.5952x; 1.2113x over previous
import functools

import numpy as np
import jax
import jax.numpy as jnp
from jax import lax
from jax.experimental import pallas as pl
from jax.experimental.pallas import tpu as pltpu
from jax.experimental.pallas import tpu_sc as plsc

_LENS = (4096, 3584, 3072, 2560, 2048, 1536, 1024, 512)
_D = 256
_TOTAL = sum(_LENS)
_NC = 2
_NS = 16
_NW = _NC * _NS


def _metadata():
    lengths = np.array(_LENS, np.int64)
    max_len = int(lengths.max())
    bs = (lengths[None, :] > np.arange(max_len)[:, None]).sum(axis=1)
    off = np.zeros(max_len, np.int64)
    off[1:] = np.cumsum(bs)[:-1]
    dests = [(off[:L] + j).astype(np.int32) for j, L in enumerate(_LENS)]
    sorted_idx = np.argsort(-lengths, kind="stable")
    unsorted_idx = np.argsort(sorted_idx)
    return bs, sorted_idx, unsorted_idx, dests


_BS, _SORTED, _UNSORTED, _DESTS = _metadata()
_CNTS = tuple(L // _NW for L in _LENS)


_SLOT_ROWS = tuple(max(_CNTS[b], _CNTS[b + 4]) for b in range(4))


def _pack_body(*refs):
    xs = refs[0:8]
    out = refs[8]
    bufs = refs[9:13]
    idxs = refs[13:21]
    gsem = refs[21:25]
    ssem = refs[25:29]
    wid = lax.axis_index("s") * _NC + lax.axis_index("c")

    def gather(j):
        cnt = _CNTS[j]
        return pltpu.async_copy(
            xs[j].at[pl.ds(wid * cnt, cnt)],
            bufs[j % 4].at[pl.ds(0, cnt)],
            gsem[j % 4],
        )

    gh = [gather(0), gather(1), None, None]
    sh = [None] * 4
    lane = lax.iota(jnp.int32, 16)
    for j in range(8):
        cnt = _CNTS[j]
        base = wid * cnt
        for c in range(cnt // 16):
            t = base + (c * 16 + lane)
            k = lax.shift_right_logical(t, 9)
            r = lax.bitwise_and(t, 511)
            dest = 256 * k * (17 - k) + r * (8 - k) + j
            idxs[j][pl.ds(c * 16, 16)] = dest
    for j in range(8):
        b = j % 4
        cnt = _CNTS[j]
        gh[b].wait()
        sh[b] = pltpu.async_copy(
            bufs[b].at[pl.ds(0, cnt)], out.at[idxs[j]], ssem[b]
        )
        if j + 2 < 8:
            if j >= 2:
                sh[(j - 2) % 4].wait()
            gh[(j + 2) % 4] = gather(j + 2)
    for j in range(4, 8):
        sh[j % 4].wait()


_pack = functools.partial(
    pl.kernel,
    mesh=plsc.VectorSubcoreMesh(core_axis_name="c", subcore_axis_name="s"),
    out_type=jax.ShapeDtypeStruct((_TOTAL, _D), jnp.float32),
    scratch_types=[pltpu.VMEM((r, _D), jnp.float32) for r in _SLOT_ROWS]
    + [pltpu.VMEM((c,), jnp.int32) for c in _CNTS]
    + [pltpu.SemaphoreType.DMA for _ in range(4)]
    + [pltpu.SemaphoreType.DMA for _ in range(4)],
)(_pack_body)


def kernel(x0, x1, x2, x3, x4, x5, x6, x7):
    xs = (x0, x1, x2, x3, x4, x5, x6, x7)
    data = _pack(*xs)
    return (
        data,
        jnp.asarray(_BS, dtype=jnp.int64),
        jnp.asarray(_SORTED, dtype=jnp.int64),
        jnp.asarray(_UNSORTED, dtype=jnp.int64),
    )
